# baseline (speedup 1.0000x reference)
import jax
import jax.numpy as jnp
from jax.experimental import pallas as pl
from jax.experimental.pallas import tpu as pltpu

_C0 = 2048
_CB = 6144


def _make_body(n_rows):
    n_mid = (n_rows - _C0) // _CB
    tail = n_rows - _C0 - n_mid * _CB
    if tail == 0:
        n_mid -= 1
        tail = _CB
    n_steps = n_mid + 2

    def off(step):
        return jnp.where(step == 0, 0, _C0 + (step - 1) * _CB)

    def body(x_ref, f_hbm, o_hbm, xn_scr, obuf, fbuf, osems, fsems):
        j = pl.program_id(0)
        slot = jax.lax.rem(j, 2)

        def f_fetch(step, size):
            return pltpu.make_async_copy(
                f_hbm.at[pl.ds(off(step), size), :],
                fbuf.at[jax.lax.rem(step, 2), pl.ds(0, size)],
                fsems.at[jax.lax.rem(step, 2)],
            )

        def o_copy(step, s, size):
            return pltpu.make_async_copy(
                obuf.at[s, pl.ds(0, size)],
                o_hbm.at[pl.ds(off(step), size), :],
                osems.at[s],
            )

        @pl.when(j == 0)
        def _():
            f_fetch(0, _C0).start()
            x = x_ref[...]
            nrm = jnp.sqrt(jnp.sum(x * x, axis=1, keepdims=True))
            xn_scr[...] = (x / jnp.maximum(nrm, 1e-12)).astype(jnp.bfloat16)
            f_fetch(1, _CB).start()
            f_fetch(0, _C0).wait()

        @pl.when((j >= 1) & (j + 1 <= n_mid))
        def _():
            f_fetch(j + 1, _CB).start()

        @pl.when(j + 1 == n_steps - 1)
        def _():
            f_fetch(j + 1, tail).start()

        @pl.when((j >= 1) & (j <= n_mid))
        def _():
            f_fetch(j, _CB).wait()

        @pl.when(j == n_steps - 1)
        def _():
            f_fetch(j, tail).wait()

        @pl.when(j == 2)
        def _():
            o_copy(0, slot, _C0).wait()

        @pl.when(j > 2)
        def _():
            o_copy(j - 2, slot, _CB).wait()

        xn = xn_scr[...]

        @pl.when(j == 0)
        def _():
            obuf[slot, :_C0] = jax.lax.dot_general(
                fbuf[0, :_C0].astype(jnp.bfloat16), xn,
                (((1,), (1,)), ((), ())), preferred_element_type=jnp.float32)
            o_copy(0, slot, _C0).start()

        @pl.when((j >= 1) & (j <= n_mid))
        def _():
            obuf[slot] = jax.lax.dot_general(
                fbuf[slot].astype(jnp.bfloat16), xn,
                (((1,), (1,)), ((), ())), preferred_element_type=jnp.float32)
            o_copy(j, slot, _CB).start()

        @pl.when(j == n_steps - 1)
        def _():
            obuf[slot, :tail] = jax.lax.dot_general(
                fbuf[jax.lax.rem(n_steps - 1, 2), :tail].astype(jnp.bfloat16),
                xn,
                (((1,), (1,)), ((), ())), preferred_element_type=jnp.float32)
            o_copy(n_steps - 1, slot, tail).start()
            o_copy(n_steps - 2, (n_steps - 2) % 2, _CB).wait()
            o_copy(n_steps - 1, (n_steps - 1) % 2, tail).wait()

    return body, n_steps


def kernel(bn_global_x, targets, features):
    b, f = bn_global_x.shape
    n = features.shape[0]
    body, n_steps = _make_body(n)
    out_t = pl.pallas_call(
        body,
        grid=(n_steps,),
        in_specs=[
            pl.BlockSpec((b, f), lambda j: (0, 0)),
            pl.BlockSpec(memory_space=pl.ANY),
        ],
        out_specs=pl.BlockSpec(memory_space=pl.ANY),
        out_shape=jax.ShapeDtypeStruct((n, b), jnp.float32),
        scratch_shapes=[
            pltpu.VMEM((b, f), jnp.bfloat16),
            pltpu.VMEM((2, _CB, b), jnp.float32),
            pltpu.VMEM((2, _CB, f), jnp.float32),
            pltpu.SemaphoreType.DMA((2,)),
            pltpu.SemaphoreType.DMA((2,)),
        ],
        compiler_params=pltpu.CompilerParams(dimension_semantics=("arbitrary",)),
    )(bn_global_x, features)
    return (out_t.T, features)
